# Initial kernel scaffold; baseline (speedup 1.0000x reference)
import jax
import jax.numpy as jnp
from jax.experimental import pallas as pl


def kernel(hidden_state, corr_index, nei_index):
    raise NotImplementedError("write your pallas kernel here")


# speedup vs baseline: 10.2134x; 10.2134x over previous
import math

import jax
import jax.numpy as jnp
from jax.experimental import pallas as pl

_EM = math.exp(-1e-6 - 1.0)


def _body(hs_ref, nei_ref, out_ref):
    mask = nei_ref[...] > 0
    mf = mask.astype(jnp.float32)
    p = jnp.float32(mf.shape[1])
    k = jnp.sum(mf, axis=1, keepdims=True)
    scale = 1.0 / (k + (p - k) * _EM)
    acc = jnp.dot(mf, hs_ref[...], preferred_element_type=jnp.float32)
    has = jnp.any(mask)
    out_ref[...] = jnp.where(has, scale * acc, hs_ref[...])


def kernel(hidden_state, corr_index, nei_index):
    del corr_index
    ped_num, m_dim = hidden_state.shape
    return pl.pallas_call(
        _body,
        out_shape=jax.ShapeDtypeStruct((ped_num, m_dim), jnp.float32),
    )(hidden_state, nei_index)
